# Initial kernel scaffold; baseline (speedup 1.0000x reference)
import jax
import jax.numpy as jnp
from jax.experimental import pallas as pl


def kernel(embeds, anchor_set_id, dists_array, W_hidden, b_hidden):
    raise NotImplementedError("write your pallas kernel here")


# speedup vs baseline: 1.8386x; 1.8386x over previous
import functools

import jax
import jax.numpy as jnp
from jax import lax
from jax.experimental import pallas as pl
from jax.experimental.pallas import tpu as pltpu
from jax.experimental.pallas import tpu_sc as plsc

N = 50000
A = 32
D = 32
BN = 2000


_sc_mesh = plsc.VectorSubcoreMesh(core_axis_name="c", subcore_axis_name="s")


@functools.partial(
    pl.kernel,
    out_type=jax.ShapeDtypeStruct((A, D), jnp.float32),
    mesh=_sc_mesh,
    scratch_types=[
        pltpu.VMEM((A,), jnp.int32),
        pltpu.VMEM((A, D), jnp.float32),
        pltpu.SemaphoreType.DMA,
    ],
)
def _sc_gather_anchors(emb_hbm, idx_hbm, out_hbm, idx_v, rows_v, sem):
    wid = lax.axis_index("s") * 2 + lax.axis_index("c")

    @pl.when(wid == 0)
    def _():
        pltpu.sync_copy(idx_hbm, idx_v)
        for half in range(2):
            vec = idx_v[pl.ds(16 * half, 16)]
            for j in range(16):
                row = vec[j]
                pltpu.sync_copy(
                    emb_hbm.at[pl.ds(row, 1), :],
                    rows_v.at[pl.ds(16 * half + j, 1), :],
                )
        pltpu.sync_copy(rows_v, out_hbm)


def _tc_body(d_ref, e_ref, a_ref, w_ref, b_ref, o_ref):
    w = w_ref[...]
    g1 = lax.dot_general(
        a_ref[...], w[:, :D], (((1,), (1,)), ((), ())),
        preferred_element_type=jnp.float32,
    ) * (1.0 / A)
    acc = jnp.dot(d_ref[...], g1, preferred_element_type=jnp.float32)
    acc = acc + lax.dot_general(
        e_ref[...], w[:, D:], (((1,), (1,)), ((), ())),
        preferred_element_type=jnp.float32,
    )
    o_ref[...] = acc + b_ref[...]


def kernel(embeds, anchor_set_id, dists_array, W_hidden, b_hidden):
    ids = anchor_set_id.astype(jnp.int32)
    anchors = _sc_gather_anchors(embeds, ids)
    b2d = b_hidden.reshape(1, D)
    out = pl.pallas_call(
        _tc_body,
        grid=(pl.cdiv(N, BN),),
        in_specs=[
            pl.BlockSpec((BN, A), lambda i: (i, 0)),
            pl.BlockSpec((BN, D), lambda i: (i, 0)),
            pl.BlockSpec((A, D), lambda i: (0, 0)),
            pl.BlockSpec((D, 2 * D), lambda i: (0, 0)),
            pl.BlockSpec((1, D), lambda i: (0, 0)),
        ],
        out_specs=pl.BlockSpec((BN, D), lambda i: (i, 0)),
        out_shape=jax.ShapeDtypeStruct((N, D), jnp.float32),
    )(dists_array, embeds, anchors, W_hidden, b2d)
    return out


# speedup vs baseline: 2.0544x; 1.1174x over previous
import functools

import jax
import jax.numpy as jnp
from jax import lax
from jax.experimental import pallas as pl
from jax.experimental.pallas import tpu as pltpu
from jax.experimental.pallas import tpu_sc as plsc

N = 50000
A = 32
D = 32
BN = 5000


_sc_mesh = plsc.VectorSubcoreMesh(core_axis_name="c", subcore_axis_name="s")


@functools.partial(
    pl.kernel,
    out_type=jax.ShapeDtypeStruct((A, D), jnp.float32),
    mesh=_sc_mesh,
    scratch_types=[
        pltpu.VMEM((A,), jnp.int32),
        pltpu.VMEM((A, D), jnp.float32),
        pltpu.SemaphoreType.DMA,
    ],
)
def _sc_gather_anchors(emb_hbm, idx_hbm, out_hbm, idx_v, rows_v, sem):
    wid = lax.axis_index("s") * 2 + lax.axis_index("c")

    @pl.when(wid == 0)
    def _():
        pltpu.sync_copy(idx_hbm, idx_v)
        copies = []
        for half in range(2):
            vec = idx_v[pl.ds(16 * half, 16)]
            for j in range(16):
                row = vec[j]
                copies.append(pltpu.async_copy(
                    emb_hbm.at[pl.ds(row, 1), :],
                    rows_v.at[pl.ds(16 * half + j, 1), :],
                    sem,
                ))
        for c in copies:
            c.wait()
        pltpu.sync_copy(rows_v, out_hbm)


def _tc_body(d_ref, e_ref, a_ref, w_ref, b_ref, o_ref):
    w = w_ref[...]
    g1 = lax.dot_general(
        a_ref[...], w[:, :D], (((1,), (1,)), ((), ())),
        preferred_element_type=jnp.float32,
    ) * (1.0 / A)
    acc = jnp.dot(d_ref[...], g1, preferred_element_type=jnp.float32)
    acc = acc + lax.dot_general(
        e_ref[...], w[:, D:], (((1,), (1,)), ((), ())),
        preferred_element_type=jnp.float32,
    )
    o_ref[...] = acc + b_ref[...]


def kernel(embeds, anchor_set_id, dists_array, W_hidden, b_hidden):
    ids = anchor_set_id.astype(jnp.int32)
    anchors = _sc_gather_anchors(embeds, ids)
    b2d = b_hidden.reshape(1, D)
    out = pl.pallas_call(
        _tc_body,
        grid=(pl.cdiv(N, BN),),
        in_specs=[
            pl.BlockSpec((BN, A), lambda i: (i, 0)),
            pl.BlockSpec((BN, D), lambda i: (i, 0)),
            pl.BlockSpec((A, D), lambda i: (0, 0)),
            pl.BlockSpec((D, 2 * D), lambda i: (0, 0)),
            pl.BlockSpec((1, D), lambda i: (0, 0)),
        ],
        out_specs=pl.BlockSpec((BN, D), lambda i: (i, 0)),
        out_shape=jax.ShapeDtypeStruct((N, D), jnp.float32),
        compiler_params=pltpu.CompilerParams(
            dimension_semantics=("arbitrary",),
        ),
    )(dists_array, embeds, anchors, W_hidden, b2d)
    return out


# speedup vs baseline: 2.1303x; 1.0370x over previous
import functools

import jax
import jax.numpy as jnp
from jax import lax
from jax.experimental import pallas as pl
from jax.experimental.pallas import tpu as pltpu
from jax.experimental.pallas import tpu_sc as plsc

N = 50000
A = 32
D = 32
BN = 10000


_sc_mesh = plsc.VectorSubcoreMesh(core_axis_name="c", subcore_axis_name="s")


@functools.partial(
    pl.kernel,
    out_type=jax.ShapeDtypeStruct((A, D), jnp.float32),
    mesh=_sc_mesh,
    scratch_types=[
        pltpu.VMEM((A,), jnp.int32),
        pltpu.VMEM((A, D), jnp.float32),
        pltpu.SemaphoreType.DMA,
    ],
)
def _sc_gather_anchors(emb_hbm, idx_hbm, out_hbm, idx_v, rows_v, sem):
    wid = lax.axis_index("s") * 2 + lax.axis_index("c")

    @pl.when(wid == 0)
    def _():
        pltpu.sync_copy(idx_hbm, idx_v)
        copies = []
        for half in range(2):
            vec = idx_v[pl.ds(16 * half, 16)]
            for j in range(16):
                row = vec[j]
                copies.append(pltpu.async_copy(
                    emb_hbm.at[pl.ds(row, 1), :],
                    rows_v.at[pl.ds(16 * half + j, 1), :],
                    sem,
                ))
        for c in copies:
            c.wait()
        pltpu.sync_copy(rows_v, out_hbm)


def _tc_body(d_ref, e_ref, a_ref, w_ref, b_ref, o_ref):
    w = w_ref[...]
    g1 = lax.dot_general(
        a_ref[...], w[:, :D], (((1,), (1,)), ((), ())),
        preferred_element_type=jnp.float32,
    ) * (1.0 / A)
    acc = jnp.dot(d_ref[...], g1, preferred_element_type=jnp.float32)
    acc = acc + lax.dot_general(
        e_ref[...], w[:, D:], (((1,), (1,)), ((), ())),
        preferred_element_type=jnp.float32,
    )
    o_ref[...] = acc + b_ref[...]


def kernel(embeds, anchor_set_id, dists_array, W_hidden, b_hidden):
    ids = anchor_set_id.astype(jnp.int32)
    anchors = _sc_gather_anchors(embeds, ids)
    b2d = b_hidden.reshape(1, D)
    out = pl.pallas_call(
        _tc_body,
        grid=(pl.cdiv(N, BN),),
        in_specs=[
            pl.BlockSpec((BN, A), lambda i: (i, 0)),
            pl.BlockSpec((BN, D), lambda i: (i, 0)),
            pl.BlockSpec((A, D), lambda i: (0, 0)),
            pl.BlockSpec((D, 2 * D), lambda i: (0, 0)),
            pl.BlockSpec((1, D), lambda i: (0, 0)),
        ],
        out_specs=pl.BlockSpec((BN, D), lambda i: (i, 0)),
        out_shape=jax.ShapeDtypeStruct((N, D), jnp.float32),
        compiler_params=pltpu.CompilerParams(
            dimension_semantics=("arbitrary",),
        ),
    )(dists_array, embeds, anchors, W_hidden, b2d)
    return out


# speedup vs baseline: 4.2304x; 1.9858x over previous
import functools

import jax
import jax.numpy as jnp
from jax import lax
from jax.experimental import pallas as pl
from jax.experimental.pallas import tpu as pltpu
from jax.experimental.pallas import tpu_sc as plsc

N = 50000
A = 32
D = 32
BL = 6400


_sc_mesh = plsc.VectorSubcoreMesh(core_axis_name="c", subcore_axis_name="s")


@functools.partial(
    pl.kernel,
    out_type=jax.ShapeDtypeStruct((D, A * 128), jnp.float32),
    mesh=_sc_mesh,
    scratch_types=[
        pltpu.VMEM((A,), jnp.int32),
        pltpu.VMEM((16 * D, 128), jnp.float32),
        pltpu.SemaphoreType.DMA,
        pltpu.SemaphoreType.DMA,
    ],
)
def _sc_gather_anchors(et_hbm, idx_hbm, out_hbm, idx_v, win_v, sem_in, sem_out):
    wid = lax.axis_index("s") * 2 + lax.axis_index("c")

    @pl.when(wid == 0)
    def _():
        pltpu.sync_copy(idx_hbm, idx_v)
        for batch in range(2):
            vec = idx_v[pl.ds(16 * batch, 16)]
            copies = []
            for k in range(16):
                base = pl.multiple_of(
                    lax.shift_left(lax.shift_right_logical(vec[k], 7), 7), 128
                )
                copies.append(pltpu.async_copy(
                    et_hbm.at[:, pl.ds(base, 128)],
                    win_v.at[pl.ds(k * D, D), :],
                    sem_in,
                ))
            for c in copies:
                c.wait()
            outs = []
            for k in range(16):
                j = 16 * batch + k
                outs.append(pltpu.async_copy(
                    win_v.at[pl.ds(k * D, D), :],
                    out_hbm.at[:, pl.ds(j * 128, 128)],
                    sem_out,
                ))
            for c in outs:
                c.wait()


def _tc_body(dt_ref, et_ref, win_ref, oh_ref, w_ref, b_ref, ot_ref):
    w = w_ref[...]
    esel_t = jnp.dot(
        win_ref[...], oh_ref[...], preferred_element_type=jnp.float32,
    )
    g1t = jnp.dot(
        w[:, :D], esel_t, preferred_element_type=jnp.float32,
    ) * (1.0 / A)
    acc = jnp.dot(g1t, dt_ref[...], preferred_element_type=jnp.float32)
    acc = acc + jnp.dot(
        w[:, D:], et_ref[...], preferred_element_type=jnp.float32,
    )
    ot_ref[...] = acc + b_ref[...]


def kernel(embeds, anchor_set_id, dists_array, W_hidden, b_hidden):
    ids = anchor_set_id.astype(jnp.int32)
    et = embeds.T
    dt = dists_array.T
    wins = _sc_gather_anchors(et, ids)
    q = jnp.arange(A, dtype=jnp.int32) * 128 + (ids & 127)
    oh = (jnp.arange(A * 128, dtype=jnp.int32)[:, None] == q[None, :]
          ).astype(jnp.float32)
    b2d = b_hidden.reshape(D, 1)
    ot = pl.pallas_call(
        _tc_body,
        grid=(pl.cdiv(N, BL),),
        in_specs=[
            pl.BlockSpec((A, BL), lambda i: (0, i)),
            pl.BlockSpec((D, BL), lambda i: (0, i)),
            pl.BlockSpec((D, A * 128), lambda i: (0, 0)),
            pl.BlockSpec((A * 128, A), lambda i: (0, 0)),
            pl.BlockSpec((D, 2 * D), lambda i: (0, 0)),
            pl.BlockSpec((D, 1), lambda i: (0, 0)),
        ],
        out_specs=pl.BlockSpec((D, BL), lambda i: (0, i)),
        out_shape=jax.ShapeDtypeStruct((D, N), jnp.float32),
        compiler_params=pltpu.CompilerParams(
            dimension_semantics=("arbitrary",),
        ),
    )(dt, et, wins, oh, W_hidden, b2d)
    return ot.T


# speedup vs baseline: 5.0402x; 1.1914x over previous
import functools

import jax
import jax.numpy as jnp
from jax import lax
from jax.experimental import pallas as pl
from jax.experimental.pallas import tpu as pltpu
from jax.experimental.pallas import tpu_sc as plsc

N = 50000
A = 32
D = 32
BL = 6400


_sc_mesh = plsc.VectorSubcoreMesh(core_axis_name="c", subcore_axis_name="s")


@functools.partial(
    pl.kernel,
    out_type=jax.ShapeDtypeStruct((D, A * 128), jnp.float32),
    mesh=_sc_mesh,
    scratch_types=[
        pltpu.VMEM((A,), jnp.int32),
        pltpu.VMEM((16 * D, 128), jnp.float32),
        pltpu.SemaphoreType.DMA,
        pltpu.SemaphoreType.DMA,
    ],
)
def _sc_gather_anchors(et_hbm, idx_hbm, out_hbm, idx_v, win_v, sem_in, sem_out):
    wid = lax.axis_index("s") * 2 + lax.axis_index("c")

    def handle(batch):
        vec = idx_v[pl.ds(16 * batch, 16)]
        copies = []
        for k in range(16):
            base = pl.multiple_of(
                lax.shift_left(lax.shift_right_logical(vec[k], 7), 7), 128
            )
            copies.append(pltpu.async_copy(
                et_hbm.at[:, pl.ds(base, 128)],
                win_v.at[pl.ds(k * D, D), :],
                sem_in,
            ))
        for c in copies:
            c.wait()
        outs = []
        for k in range(16):
            j = 16 * batch + k
            outs.append(pltpu.async_copy(
                win_v.at[pl.ds(k * D, D), :],
                out_hbm.at[:, pl.ds(j * 128, 128)],
                sem_out,
            ))
        for c in outs:
            c.wait()

    @pl.when(wid == 0)
    def _():
        pltpu.sync_copy(idx_hbm, idx_v)
        handle(0)

    @pl.when(wid == 1)
    def _():
        pltpu.sync_copy(idx_hbm, idx_v)
        handle(1)


def _tc_body(dt_ref, et_ref, win_ref, oh_ref, w_ref, b_ref, ot_ref, g1t_ref):
    w = w_ref[...]

    @pl.when(pl.program_id(0) == 0)
    def _():
        esel_t = jnp.dot(
            win_ref[...], oh_ref[...], preferred_element_type=jnp.float32,
        )
        g1t_ref[...] = jnp.dot(
            w[:, :D], esel_t, preferred_element_type=jnp.float32,
        ) * (1.0 / A)

    acc = jnp.dot(g1t_ref[...], dt_ref[...], preferred_element_type=jnp.float32)
    acc = acc + jnp.dot(
        w[:, D:], et_ref[...], preferred_element_type=jnp.float32,
    )
    ot_ref[...] = acc + b_ref[...]


def kernel(embeds, anchor_set_id, dists_array, W_hidden, b_hidden):
    ids = anchor_set_id.astype(jnp.int32)
    et = embeds.T
    dt = dists_array.T
    wins = _sc_gather_anchors(et, ids)
    q = jnp.arange(A, dtype=jnp.int32) * 128 + (ids & 127)
    oh = (jnp.arange(A * 128, dtype=jnp.int32)[:, None] == q[None, :]
          ).astype(jnp.float32)
    b2d = b_hidden.reshape(D, 1)
    ot = pl.pallas_call(
        _tc_body,
        grid=(pl.cdiv(N, BL),),
        in_specs=[
            pl.BlockSpec((A, BL), lambda i: (0, i)),
            pl.BlockSpec((D, BL), lambda i: (0, i)),
            pl.BlockSpec((D, A * 128), lambda i: (0, 0)),
            pl.BlockSpec((A * 128, A), lambda i: (0, 0)),
            pl.BlockSpec((D, 2 * D), lambda i: (0, 0)),
            pl.BlockSpec((D, 1), lambda i: (0, 0)),
        ],
        out_specs=pl.BlockSpec((D, BL), lambda i: (0, i)),
        out_shape=jax.ShapeDtypeStruct((D, N), jnp.float32),
        scratch_shapes=[pltpu.VMEM((D, A), jnp.float32)],
        compiler_params=pltpu.CompilerParams(
            dimension_semantics=("arbitrary",),
        ),
    )(dt, et, wins, oh, W_hidden, b2d)
    return ot.T


# speedup vs baseline: 5.3731x; 1.0660x over previous
import functools

import jax
import jax.numpy as jnp
from jax import lax
from jax.experimental import pallas as pl
from jax.experimental.pallas import tpu as pltpu
from jax.experimental.pallas import tpu_sc as plsc

N = 50000
A = 32
D = 32
BL = 12800


_sc_mesh = plsc.VectorSubcoreMesh(core_axis_name="c", subcore_axis_name="s")


@functools.partial(
    pl.kernel,
    out_type=jax.ShapeDtypeStruct((D, A * 128), jnp.float32),
    mesh=_sc_mesh,
    scratch_types=[
        pltpu.VMEM((A,), jnp.int32),
        pltpu.VMEM((16 * D, 128), jnp.float32),
        pltpu.SemaphoreType.DMA,
        pltpu.SemaphoreType.DMA,
    ],
)
def _sc_gather_anchors(et_hbm, idx_hbm, out_hbm, idx_v, win_v, sem_in, sem_out):
    wid = lax.axis_index("s") * 2 + lax.axis_index("c")

    def handle(batch):
        vec = idx_v[pl.ds(16 * batch, 16)]
        copies = []
        for k in range(16):
            base = pl.multiple_of(
                lax.shift_left(lax.shift_right_logical(vec[k], 7), 7), 128
            )
            copies.append(pltpu.async_copy(
                et_hbm.at[:, pl.ds(base, 128)],
                win_v.at[pl.ds(k * D, D), :],
                sem_in,
            ))
        for c in copies:
            c.wait()
        outs = []
        for k in range(16):
            j = 16 * batch + k
            outs.append(pltpu.async_copy(
                win_v.at[pl.ds(k * D, D), :],
                out_hbm.at[:, pl.ds(j * 128, 128)],
                sem_out,
            ))
        for c in outs:
            c.wait()

    @pl.when(wid == 0)
    def _():
        pltpu.sync_copy(idx_hbm, idx_v)
        handle(0)

    @pl.when(wid == 1)
    def _():
        pltpu.sync_copy(idx_hbm, idx_v)
        handle(1)


def _tc_body(dt_ref, et_ref, win_ref, oh_ref, w_ref, b_ref, ot_ref, g1t_ref):
    w = w_ref[...]

    @pl.when(pl.program_id(0) == 0)
    def _():
        esel_t = jnp.dot(
            win_ref[...], oh_ref[...], preferred_element_type=jnp.float32,
        )
        g1t_ref[...] = jnp.dot(
            w[:, :D], esel_t, preferred_element_type=jnp.float32,
        ) * (1.0 / A)

    acc = jnp.dot(g1t_ref[...], dt_ref[...], preferred_element_type=jnp.float32)
    acc = acc + jnp.dot(
        w[:, D:], et_ref[...], preferred_element_type=jnp.float32,
    )
    ot_ref[...] = acc + b_ref[...]


def kernel(embeds, anchor_set_id, dists_array, W_hidden, b_hidden):
    ids = anchor_set_id.astype(jnp.int32)
    et = embeds.T
    dt = dists_array.T
    wins = _sc_gather_anchors(et, ids)
    q = jnp.arange(A, dtype=jnp.int32) * 128 + (ids & 127)
    oh = (jnp.arange(A * 128, dtype=jnp.int32)[:, None] == q[None, :]
          ).astype(jnp.float32)
    b2d = b_hidden.reshape(D, 1)
    ot = pl.pallas_call(
        _tc_body,
        grid=(pl.cdiv(N, BL),),
        in_specs=[
            pl.BlockSpec((A, BL), lambda i: (0, i)),
            pl.BlockSpec((D, BL), lambda i: (0, i)),
            pl.BlockSpec((D, A * 128), lambda i: (0, 0)),
            pl.BlockSpec((A * 128, A), lambda i: (0, 0)),
            pl.BlockSpec((D, 2 * D), lambda i: (0, 0)),
            pl.BlockSpec((D, 1), lambda i: (0, 0)),
        ],
        out_specs=pl.BlockSpec((D, BL), lambda i: (0, i)),
        out_shape=jax.ShapeDtypeStruct((D, N), jnp.float32),
        scratch_shapes=[pltpu.VMEM((D, A), jnp.float32)],
        compiler_params=pltpu.CompilerParams(
            dimension_semantics=("arbitrary",),
        ),
    )(dt, et, wins, oh, W_hidden, b2d)
    return ot.T


# speedup vs baseline: 5.6317x; 1.0481x over previous
import functools

import jax
import jax.numpy as jnp
from jax import lax
from jax.experimental import pallas as pl
from jax.experimental.pallas import tpu as pltpu
from jax.experimental.pallas import tpu_sc as plsc

N = 50000
A = 32
D = 32
BL = 25600


_sc_mesh = plsc.VectorSubcoreMesh(core_axis_name="c", subcore_axis_name="s")


@functools.partial(
    pl.kernel,
    out_type=jax.ShapeDtypeStruct((D, A * 128), jnp.float32),
    mesh=_sc_mesh,
    scratch_types=[
        pltpu.VMEM((A,), jnp.int32),
        pltpu.VMEM((16 * D, 128), jnp.float32),
        pltpu.SemaphoreType.DMA,
        pltpu.SemaphoreType.DMA,
    ],
)
def _sc_gather_anchors(et_hbm, idx_hbm, out_hbm, idx_v, win_v, sem_in, sem_out):
    wid = lax.axis_index("s") * 2 + lax.axis_index("c")

    def handle(batch):
        vec = idx_v[pl.ds(16 * batch, 16)]
        copies = []
        for k in range(16):
            base = pl.multiple_of(
                lax.shift_left(lax.shift_right_logical(vec[k], 7), 7), 128
            )
            copies.append(pltpu.async_copy(
                et_hbm.at[:, pl.ds(base, 128)],
                win_v.at[pl.ds(k * D, D), :],
                sem_in,
            ))
        for c in copies:
            c.wait()
        outs = []
        for k in range(16):
            j = 16 * batch + k
            outs.append(pltpu.async_copy(
                win_v.at[pl.ds(k * D, D), :],
                out_hbm.at[:, pl.ds(j * 128, 128)],
                sem_out,
            ))
        for c in outs:
            c.wait()

    @pl.when(wid == 0)
    def _():
        pltpu.sync_copy(idx_hbm, idx_v)
        handle(0)

    @pl.when(wid == 1)
    def _():
        pltpu.sync_copy(idx_hbm, idx_v)
        handle(1)


def _tc_body(dt_ref, et_ref, win_ref, oh_ref, w_ref, b_ref, ot_ref, g1t_ref):
    w = w_ref[...]

    @pl.when(pl.program_id(0) == 0)
    def _():
        esel_t = jnp.dot(
            win_ref[...], oh_ref[...], preferred_element_type=jnp.float32,
        )
        g1t_ref[...] = jnp.dot(
            w[:, :D], esel_t, preferred_element_type=jnp.float32,
        ) * (1.0 / A)

    acc = jnp.dot(g1t_ref[...], dt_ref[...], preferred_element_type=jnp.float32)
    acc = acc + jnp.dot(
        w[:, D:], et_ref[...], preferred_element_type=jnp.float32,
    )
    ot_ref[...] = acc + b_ref[...]


def kernel(embeds, anchor_set_id, dists_array, W_hidden, b_hidden):
    ids = anchor_set_id.astype(jnp.int32)
    et = embeds.T
    dt = dists_array.T
    wins = _sc_gather_anchors(et, ids)
    q = jnp.arange(A, dtype=jnp.int32) * 128 + (ids & 127)
    oh = (jnp.arange(A * 128, dtype=jnp.int32)[:, None] == q[None, :]
          ).astype(jnp.float32)
    b2d = b_hidden.reshape(D, 1)
    ot = pl.pallas_call(
        _tc_body,
        grid=(pl.cdiv(N, BL),),
        in_specs=[
            pl.BlockSpec((A, BL), lambda i: (0, i)),
            pl.BlockSpec((D, BL), lambda i: (0, i)),
            pl.BlockSpec((D, A * 128), lambda i: (0, 0)),
            pl.BlockSpec((A * 128, A), lambda i: (0, 0)),
            pl.BlockSpec((D, 2 * D), lambda i: (0, 0)),
            pl.BlockSpec((D, 1), lambda i: (0, 0)),
        ],
        out_specs=pl.BlockSpec((D, BL), lambda i: (0, i)),
        out_shape=jax.ShapeDtypeStruct((D, N), jnp.float32),
        scratch_shapes=[pltpu.VMEM((D, A), jnp.float32)],
        compiler_params=pltpu.CompilerParams(
            dimension_semantics=("arbitrary",),
        ),
    )(dt, et, wins, oh, W_hidden, b2d)
    return ot.T


# speedup vs baseline: 6.3701x; 1.1311x over previous
import functools

import jax
import jax.numpy as jnp
from jax import lax
from jax.experimental import pallas as pl
from jax.experimental.pallas import tpu as pltpu
from jax.experimental.pallas import tpu_sc as plsc

N = 50000
A = 32
D = 32
BL = 25600


_sc_mesh = plsc.VectorSubcoreMesh(core_axis_name="c", subcore_axis_name="s")


@functools.partial(
    pl.kernel,
    out_type=jax.ShapeDtypeStruct((D, A * 128), jnp.float32),
    mesh=_sc_mesh,
    scratch_types=[
        pltpu.VMEM((A,), jnp.int32),
        pltpu.VMEM((16 * D, 128), jnp.float32),
        pltpu.SemaphoreType.DMA,
        pltpu.SemaphoreType.DMA,
    ],
)
def _sc_gather_anchors(et_hbm, idx_hbm, out_hbm, idx_v, win_v, sem_in, sem_out):
    wid = lax.axis_index("s") * 2 + lax.axis_index("c")

    PER = A // 32

    def handle(w):
        j0 = PER * w
        vec = idx_v[pl.ds(16 * (j0 // 16), 16)]
        copies = []
        for t in range(PER):
            k = (j0 + t) % 16
            base = pl.multiple_of(
                lax.shift_left(lax.shift_right_logical(vec[k], 7), 7), 128
            )
            copies.append(pltpu.async_copy(
                et_hbm.at[:, pl.ds(base, 128)],
                win_v.at[pl.ds(t * D, D), :],
                sem_in,
            ))
        for c in copies:
            c.wait()
        outs = []
        for t in range(PER):
            j = j0 + t
            outs.append(pltpu.async_copy(
                win_v.at[pl.ds(t * D, D), :],
                out_hbm.at[:, pl.ds(j * 128, 128)],
                sem_out,
            ))
        for c in outs:
            c.wait()

    for _w in range(32):
        @pl.when(wid == _w)
        def _(w=_w):
            pltpu.sync_copy(idx_hbm, idx_v)
            handle(w)


def _tc_body(dt_ref, et_ref, win_ref, ids_ref, w_ref, b_ref, ot_ref, g1t_ref):
    w = w_ref[...]

    @pl.when(pl.program_id(0) == 0)
    def _():
        r = lax.broadcasted_iota(jnp.int32, (A * 128, A), 0)
        a_idx = lax.broadcasted_iota(jnp.int32, (A * 128, A), 1)
        off = jnp.broadcast_to(
            lax.bitwise_and(ids_ref[...], 127), (A * 128, A)
        )
        oh = (r == a_idx * 128 + off).astype(jnp.float32)
        esel_t = jnp.dot(
            win_ref[...], oh, preferred_element_type=jnp.float32,
        )
        g1t_ref[...] = jnp.dot(
            w[:, :D], esel_t, preferred_element_type=jnp.float32,
        ) * (1.0 / A)

    acc = jnp.dot(
        g1t_ref[...], dt_ref[...],
        preferred_element_type=jnp.float32,
        precision=lax.Precision.DEFAULT,
    )
    acc = acc + jnp.dot(
        w[:, D:], et_ref[...],
        preferred_element_type=jnp.float32,
        precision=lax.Precision.DEFAULT,
    )
    ot_ref[...] = acc + b_ref[...]


def kernel(embeds, anchor_set_id, dists_array, W_hidden, b_hidden):
    ids = anchor_set_id.astype(jnp.int32)
    et = embeds.T
    dt = dists_array.T
    wins = _sc_gather_anchors(et, ids)
    ids2d = ids.reshape(1, A)
    b2d = b_hidden.reshape(D, 1)
    ot = pl.pallas_call(
        _tc_body,
        grid=(pl.cdiv(N, BL),),
        in_specs=[
            pl.BlockSpec((A, BL), lambda i: (0, i)),
            pl.BlockSpec((D, BL), lambda i: (0, i)),
            pl.BlockSpec((D, A * 128), lambda i: (0, 0)),
            pl.BlockSpec((1, A), lambda i: (0, 0)),
            pl.BlockSpec((D, 2 * D), lambda i: (0, 0)),
            pl.BlockSpec((D, 1), lambda i: (0, 0)),
        ],
        out_specs=pl.BlockSpec((D, BL), lambda i: (0, i)),
        out_shape=jax.ShapeDtypeStruct((D, N), jnp.float32),
        scratch_shapes=[pltpu.VMEM((D, A), jnp.float32)],
        compiler_params=pltpu.CompilerParams(
            dimension_semantics=("arbitrary",),
        ),
    )(dt, et, wins, ids2d, W_hidden, b2d)
    return ot.T


# speedup vs baseline: 6.9544x; 1.0917x over previous
import functools

import jax
import jax.numpy as jnp
from jax import lax
from jax.experimental import pallas as pl
from jax.experimental.pallas import tpu as pltpu
from jax.experimental.pallas import tpu_sc as plsc

N = 50000
A = 32
D = 32
BL = 25600


_sc_mesh = plsc.VectorSubcoreMesh(core_axis_name="c", subcore_axis_name="s", num_cores=1)


@functools.partial(
    pl.kernel,
    out_type=jax.ShapeDtypeStruct((D, A * 128), jnp.float32),
    mesh=_sc_mesh,
    scratch_types=[
        pltpu.VMEM((A,), jnp.int32),
        pltpu.VMEM((16 * D, 128), jnp.float32),
        pltpu.SemaphoreType.DMA,
        pltpu.SemaphoreType.DMA,
    ],
)
def _sc_gather_anchors(et_hbm, idx_hbm, out_hbm, idx_v, win_v, sem_in, sem_out):
    wid = lax.axis_index("s") * 2 + lax.axis_index("c")

    PER = A // 16

    def handle(w):
        j0 = PER * w
        vec = idx_v[pl.ds(16 * (j0 // 16), 16)]
        copies = []
        for t in range(PER):
            k = (j0 + t) % 16
            base = pl.multiple_of(
                lax.shift_left(lax.shift_right_logical(vec[k], 7), 7), 128
            )
            copies.append(pltpu.async_copy(
                et_hbm.at[:, pl.ds(base, 128)],
                win_v.at[pl.ds(t * D, D), :],
                sem_in,
            ))
        for c in copies:
            c.wait()
        outs = []
        for t in range(PER):
            j = j0 + t
            outs.append(pltpu.async_copy(
                win_v.at[pl.ds(t * D, D), :],
                out_hbm.at[:, pl.ds(j * 128, 128)],
                sem_out,
            ))
        for c in outs:
            c.wait()

    for _w in range(16):
        @pl.when(wid == _w)
        def _(w=_w):
            pltpu.sync_copy(idx_hbm, idx_v)
            handle(w)


def _tc_body(dt_ref, et_ref, win_ref, ids_ref, w_ref, b_ref, ot_ref, g1t_ref):
    w = w_ref[...]

    @pl.when(pl.program_id(0) == 0)
    def _():
        r = lax.broadcasted_iota(jnp.int32, (A * 128, A), 0)
        a_idx = lax.broadcasted_iota(jnp.int32, (A * 128, A), 1)
        off = jnp.broadcast_to(
            lax.bitwise_and(ids_ref[...], 127), (A * 128, A)
        )
        oh = (r == a_idx * 128 + off).astype(jnp.float32)
        esel_t = jnp.dot(
            win_ref[...], oh, preferred_element_type=jnp.float32,
        )
        g1t_ref[...] = jnp.dot(
            w[:, :D], esel_t, preferred_element_type=jnp.float32,
        ) * (1.0 / A)

    acc = jnp.dot(
        g1t_ref[...], dt_ref[...],
        preferred_element_type=jnp.float32,
        precision=lax.Precision.DEFAULT,
    )
    acc = acc + jnp.dot(
        w[:, D:], et_ref[...],
        preferred_element_type=jnp.float32,
        precision=lax.Precision.DEFAULT,
    )
    ot_ref[...] = acc + b_ref[...]


def kernel(embeds, anchor_set_id, dists_array, W_hidden, b_hidden):
    ids = anchor_set_id.astype(jnp.int32)
    et = embeds.T
    dt = dists_array.T
    wins = _sc_gather_anchors(et, ids)
    ids2d = ids.reshape(1, A)
    b2d = b_hidden.reshape(D, 1)
    ot = pl.pallas_call(
        _tc_body,
        grid=(pl.cdiv(N, BL),),
        in_specs=[
            pl.BlockSpec((A, BL), lambda i: (0, i)),
            pl.BlockSpec((D, BL), lambda i: (0, i)),
            pl.BlockSpec((D, A * 128), lambda i: (0, 0)),
            pl.BlockSpec((1, A), lambda i: (0, 0)),
            pl.BlockSpec((D, 2 * D), lambda i: (0, 0)),
            pl.BlockSpec((D, 1), lambda i: (0, 0)),
        ],
        out_specs=pl.BlockSpec((D, BL), lambda i: (0, i)),
        out_shape=jax.ShapeDtypeStruct((D, N), jnp.float32),
        scratch_shapes=[pltpu.VMEM((D, A), jnp.float32)],
        compiler_params=pltpu.CompilerParams(
            dimension_semantics=("arbitrary",),
        ),
    )(dt, et, wins, ids2d, W_hidden, b2d)
    return ot.T
